# baseline (speedup 1.0000x reference)
import functools

import jax
import jax.numpy as jnp
from jax import lax
from jax.experimental import pallas as pl
from jax.experimental.pallas import tpu as pltpu
from jax.experimental.pallas import tpu_sc as plsc

N = 100000
D = 128
G = 512
S = 2 * G
TRASH = S
S2 = S + 128
NC, NS = 2, 16
NW = NC * NS
L = 16
GRP = 128
FULL = N // GRP
TAIL = N - FULL * GRP
LASTBASE = N - GRP
VS_TAIL = GRP - TAIL
NGRP = FULL + (1 if TAIL else 0)
K = (NGRP + NW - 1) // NW
ZR = S2 // NS
ZB = 24
NBUF = 3


def _sc_body(out_hbm, z_hbm, b_hbm, part_hbm, cnt_hbm,
             acc, rows0, rows1, rows2, zb0, zb1, zb2, idx_v, cnt_v,
             cnt_red, zero_v, sem0, sem1, sem2, zsem):
    cid = lax.axis_index("c")
    sid = lax.axis_index("s")
    wid = sid * NC + cid
    lanes = lax.iota(jnp.int32, L)
    ones = jnp.ones((L,), jnp.float32)

    bufs = ((zb0, rows0, sem0), (zb1, rows1, sem1), (zb2, rows2, sem2))

    def _base(k):
        g = wid + NW * k
        return g, pl.multiple_of(
            jnp.where(g < FULL, g * GRP, jnp.int32(LASTBASE)), 8)

    def _start(k, buf):
        zb_b, rows_b, sem_b = buf
        _, base = _base(k)
        pltpu.async_copy(z_hbm.at[pl.ds(base, GRP)], zb_b.at[0], sem_b)
        pltpu.async_copy(b_hbm.at[pl.ds(base, GRP)], zb_b.at[1], sem_b)
        pltpu.async_copy(out_hbm.at[pl.ds(base, GRP)], rows_b, sem_b)

    for b in range(NBUF):
        _start(b, bufs[b])

    def _fill_zero(i, _):
        for j in range(D // L):
            zero_v[i, pl.ds(j * L, L)] = jnp.zeros((L,), jnp.float32)
        return 0
    lax.fori_loop(0, ZB, _fill_zero, 0)

    def _fill_cnt(i, _):
        for r in range(L):
            cnt_v[r, pl.ds(i * L, L)] = jnp.zeros((L,), jnp.float32)
        return 0
    lax.fori_loop(0, S2 // L, _fill_cnt, 0)

    for t in range(ZR // ZB):
        pltpu.async_copy(zero_v, acc.at[pl.ds(sid * ZR + t * ZB, ZB)],
                         zsem)
    for t in range(ZR // ZB):
        pltpu.make_async_copy(zero_v, acc.at[pl.ds(t * ZB, ZB)],
                              zsem).wait()
    plsc.subcore_barrier()

    def _wait_inputs(buf):
        zb_b, rows_b, sem_b = buf
        pltpu.make_async_copy(z_hbm.at[pl.ds(0, GRP)], zb_b.at[0],
                              sem_b).wait()
        pltpu.make_async_copy(b_hbm.at[pl.ds(0, GRP)], zb_b.at[1],
                              sem_b).wait()
        pltpu.make_async_copy(out_hbm.at[pl.ds(0, GRP)], rows_b,
                              sem_b).wait()

    def _indices(k, buf):
        zb_b, _, _ = buf
        g = wid + NW * k
        vs = jnp.where(g < FULL, jnp.int32(0),
                       jnp.where(g == FULL, jnp.int32(VS_TAIL),
                                 jnp.int32(GRP)))
        for j in range(GRP // L):
            zz = zb_b[0, pl.ds(j * L, L)]
            bb = zb_b[1, pl.ds(j * L, L)]
            slot = bb + jnp.where(zz == jnp.int32(100), jnp.int32(G),
                                  jnp.int32(0))
            slot = jnp.where(j * L + lanes >= vs, slot, jnp.int32(TRASH))
            idx_v[0, pl.ds(j * L, L)] = slot
            plsc.addupdate_scatter(cnt_v, [lanes, slot], ones)

    def _round(kk, _):
        for b in range(NBUF):
            k = NBUF * kk + b
            buf = bufs[b]

            @pl.when(k < K)
            def _():
                _wait_inputs(buf)
                _indices(k, buf)
                pltpu.sync_copy(buf[1], acc.at[idx_v.at[0]], add=True)

                @pl.when(k + NBUF < K)
                def _():
                    _start(k + NBUF, buf)
        return 0
    lax.fori_loop(0, (K + NBUF - 1) // NBUF, _round, 0)

    plsc.subcore_barrier()
    def _red_cnt(i, _):
        tot = cnt_v[0, pl.ds(i * L, L)]
        for r in range(1, L):
            tot = tot + cnt_v[r, pl.ds(i * L, L)]
        cnt_red[pl.ds(i * L, L)] = tot
        return 0
    lax.fori_loop(0, S2 // L, _red_cnt, 0)
    pltpu.sync_copy(acc.at[pl.ds(sid * ZR, ZR)],
                    part_hbm.at[cid, pl.ds(sid * ZR, ZR)])
    pltpu.sync_copy(cnt_red, cnt_hbm.at[wid])


_sc_pool = functools.partial(
    pl.kernel,
    out_type=(jax.ShapeDtypeStruct((NC, S2, D), jnp.float32),
              jax.ShapeDtypeStruct((NW, S2), jnp.float32)),
    mesh=plsc.VectorSubcoreMesh(core_axis_name="c", subcore_axis_name="s",
                                num_cores=NC, num_subcores=NS),
    compiler_params=pltpu.CompilerParams(needs_layout_passes=False),
    scratch_types=[
        pltpu.VMEM_SHARED((S2, D), jnp.float32),
        pltpu.VMEM((GRP, D), jnp.float32),
        pltpu.VMEM((GRP, D), jnp.float32),
        pltpu.VMEM((GRP, D), jnp.float32),
        pltpu.VMEM((2, GRP), jnp.int32),
        pltpu.VMEM((2, GRP), jnp.int32),
        pltpu.VMEM((2, GRP), jnp.int32),
        pltpu.VMEM((1, GRP), jnp.int32),
        pltpu.VMEM((L, S2), jnp.float32),
        pltpu.VMEM((S2,), jnp.float32),
        pltpu.VMEM((ZB, D), jnp.float32),
        pltpu.SemaphoreType.DMA,
        pltpu.SemaphoreType.DMA,
        pltpu.SemaphoreType.DMA,
        pltpu.SemaphoreType.DMA,
    ],
)(_sc_body)


def _fin_body(p_ref, c_ref, o_ref):
    sums = p_ref[0] + p_ref[1]
    cnts = jnp.sum(c_ref[...], axis=0)
    denom = jnp.maximum(cnts, 1.0)[:, None]
    means = sums / denom
    o_ref[:, :D] = means[:G]
    o_ref[:, D:] = means[G:S]


def kernel(out, z, batch):
    part, cnts = _sc_pool(out, z.astype(jnp.int32), batch.astype(jnp.int32))
    return pl.pallas_call(
        _fin_body,
        out_shape=jax.ShapeDtypeStruct((G, 2 * D), jnp.float32),
    )(part, cnts)


# speedup vs baseline: 1.0258x; 1.0258x over previous
import functools

import jax
import jax.numpy as jnp
from jax import lax
from jax.experimental import pallas as pl
from jax.experimental.pallas import tpu as pltpu
from jax.experimental.pallas import tpu_sc as plsc

N = 100000
D = 128
G = 512
S = 2 * G
TRASH = S
S2 = S + 128
NC, NS = 2, 16
NW = NC * NS
L = 16
GRP = 128
FULL = N // GRP
TAIL = N - FULL * GRP
LASTBASE = N - GRP
VS_TAIL = GRP - TAIL
NGRP = FULL + (1 if TAIL else 0)
K = (NGRP + NW - 1) // NW
ZR = S2 // NS
ZB = 24
NBUF = 3


def _sc_body(out_hbm, z_hbm, b_hbm, part_hbm, cnt_hbm,
             acc, rows0, rows1, rows2, zb0, zb1, zb2, idx0, idx1, idx2,
             cnt_v, cnt_red, zero_v, sem0, sem1, sem2, ssem0, ssem1, ssem2,
             zsem):
    cid = lax.axis_index("c")
    sid = lax.axis_index("s")
    wid = sid * NC + cid
    lanes = lax.iota(jnp.int32, L)
    ones = jnp.ones((L,), jnp.float32)

    bufs = ((zb0, rows0, sem0, idx0, ssem0),
            (zb1, rows1, sem1, idx1, ssem1),
            (zb2, rows2, sem2, idx2, ssem2))

    def _base(k):
        g = wid + NW * k
        return g, pl.multiple_of(
            jnp.where(g < FULL, g * GRP, jnp.int32(LASTBASE)), 8)

    def _start(k, buf):
        zb_b, rows_b, sem_b = buf[0], buf[1], buf[2]
        _, base = _base(k)
        pltpu.async_copy(z_hbm.at[pl.ds(base, GRP)], zb_b.at[0], sem_b)
        pltpu.async_copy(b_hbm.at[pl.ds(base, GRP)], zb_b.at[1], sem_b)
        pltpu.async_copy(out_hbm.at[pl.ds(base, GRP)], rows_b, sem_b)

    for b in range(NBUF):
        _start(b, bufs[b])

    def _fill_zero(i, _):
        for j in range(D // L):
            zero_v[i, pl.ds(j * L, L)] = jnp.zeros((L,), jnp.float32)
        return 0
    lax.fori_loop(0, ZB, _fill_zero, 0)

    def _fill_cnt(i, _):
        for r in range(L):
            cnt_v[r, pl.ds(i * L, L)] = jnp.zeros((L,), jnp.float32)
        return 0
    lax.fori_loop(0, S2 // L, _fill_cnt, 0)

    for t in range(ZR // ZB):
        pltpu.async_copy(zero_v, acc.at[pl.ds(sid * ZR + t * ZB, ZB)],
                         zsem)
    for t in range(ZR // ZB):
        pltpu.make_async_copy(zero_v, acc.at[pl.ds(t * ZB, ZB)],
                              zsem).wait()
    plsc.subcore_barrier()

    def _wait_inputs(buf):
        zb_b, rows_b, sem_b = buf[0], buf[1], buf[2]
        pltpu.make_async_copy(z_hbm.at[pl.ds(0, GRP)], zb_b.at[0],
                              sem_b).wait()
        pltpu.make_async_copy(b_hbm.at[pl.ds(0, GRP)], zb_b.at[1],
                              sem_b).wait()
        pltpu.make_async_copy(out_hbm.at[pl.ds(0, GRP)], rows_b,
                              sem_b).wait()

    def _indices(k, buf):
        zb_b, idx_b = buf[0], buf[3]
        g = wid + NW * k
        vs = jnp.where(g < FULL, jnp.int32(0),
                       jnp.where(g == FULL, jnp.int32(VS_TAIL),
                                 jnp.int32(GRP)))
        for j in range(GRP // L):
            zz = zb_b[0, pl.ds(j * L, L)]
            bb = zb_b[1, pl.ds(j * L, L)]
            slot = bb + jnp.where(zz == jnp.int32(100), jnp.int32(G),
                                  jnp.int32(0))
            slot = jnp.where(j * L + lanes >= vs, slot, jnp.int32(TRASH))
            idx_b[0, pl.ds(j * L, L)] = slot
            plsc.addupdate_scatter(cnt_v, [lanes, slot], ones)

    def _wait_scatter(buf):
        _, rows_b, _, idx_b, ssem_b = buf
        pltpu.make_async_copy(rows_b, acc.at[idx_b.at[0]], ssem_b).wait()

    def _round(kk, _):
        for b in range(NBUF):
            k = NBUF * kk + b
            buf = bufs[b]
            prev = bufs[(b + NBUF - 1) % NBUF]

            @pl.when(k < K)
            def _():
                _wait_inputs(buf)
                _indices(k, buf)

                @pl.when(k >= 1)
                def _():
                    _wait_scatter(prev)

                @pl.when((k >= 1) & (k + NBUF - 1 < K))
                def _():
                    _start(k + NBUF - 1, prev)
                pltpu.async_copy(buf[1], acc.at[buf[3].at[0]], buf[4],
                                 add=True)
        return 0
    lax.fori_loop(0, (K + NBUF - 1) // NBUF, _round, 0)
    _wait_scatter(bufs[(K - 1) % NBUF])

    plsc.subcore_barrier()
    def _red_cnt(i, _):
        tot = cnt_v[0, pl.ds(i * L, L)]
        for r in range(1, L):
            tot = tot + cnt_v[r, pl.ds(i * L, L)]
        cnt_red[pl.ds(i * L, L)] = tot
        return 0
    lax.fori_loop(0, S2 // L, _red_cnt, 0)
    pltpu.sync_copy(acc.at[pl.ds(sid * ZR, ZR)],
                    part_hbm.at[cid, pl.ds(sid * ZR, ZR)])
    pltpu.sync_copy(cnt_red, cnt_hbm.at[wid])


_sc_pool = functools.partial(
    pl.kernel,
    out_type=(jax.ShapeDtypeStruct((NC, S2, D), jnp.float32),
              jax.ShapeDtypeStruct((NW, S2), jnp.float32)),
    mesh=plsc.VectorSubcoreMesh(core_axis_name="c", subcore_axis_name="s",
                                num_cores=NC, num_subcores=NS),
    compiler_params=pltpu.CompilerParams(needs_layout_passes=False),
    scratch_types=[
        pltpu.VMEM_SHARED((S2, D), jnp.float32),
        pltpu.VMEM((GRP, D), jnp.float32),
        pltpu.VMEM((GRP, D), jnp.float32),
        pltpu.VMEM((GRP, D), jnp.float32),
        pltpu.VMEM((2, GRP), jnp.int32),
        pltpu.VMEM((2, GRP), jnp.int32),
        pltpu.VMEM((2, GRP), jnp.int32),
        pltpu.VMEM((1, GRP), jnp.int32),
        pltpu.VMEM((1, GRP), jnp.int32),
        pltpu.VMEM((1, GRP), jnp.int32),
        pltpu.VMEM((L, S2), jnp.float32),
        pltpu.VMEM((S2,), jnp.float32),
        pltpu.VMEM((ZB, D), jnp.float32),
        pltpu.SemaphoreType.DMA,
        pltpu.SemaphoreType.DMA,
        pltpu.SemaphoreType.DMA,
        pltpu.SemaphoreType.DMA,
        pltpu.SemaphoreType.DMA,
        pltpu.SemaphoreType.DMA,
        pltpu.SemaphoreType.DMA,
    ],
)(_sc_body)


def _fin_body(p_ref, c_ref, o_ref):
    sums = p_ref[0] + p_ref[1]
    cnts = jnp.sum(c_ref[...], axis=0)
    denom = jnp.maximum(cnts, 1.0)[:, None]
    means = sums / denom
    o_ref[:, :D] = means[:G]
    o_ref[:, D:] = means[G:S]


def kernel(out, z, batch):
    part, cnts = _sc_pool(out, z.astype(jnp.int32), batch.astype(jnp.int32))
    return pl.pallas_call(
        _fin_body,
        out_shape=jax.ShapeDtypeStruct((G, 2 * D), jnp.float32),
    )(part, cnts)
